# baseline (speedup 1.0000x reference)
import functools

import jax
import jax.numpy as jnp
from jax import lax
from jax.experimental import pallas as pl
from jax.experimental.pallas import tpu as pltpu
from jax.experimental.pallas import tpu_sc as plsc

CHUNK = 128


def _sc_geometry():
    try:
        info = plsc.get_sparse_core_info()
        return info.num_cores, info.num_subcores
    except Exception:
        return 2, 16


@functools.lru_cache(maxsize=None)
def _build(B, V, D, nc, ns):
    nw = nc * ns
    b_per_w = B // nw
    n_chunks = b_per_w // CHUNK
    mesh = plsc.VectorSubcoreMesh(core_axis_name="c", subcore_axis_name="s")

    @functools.partial(
        pl.kernel,
        mesh=mesh,
        out_type=jax.ShapeDtypeStruct((B, D), jnp.float32),
        scratch_types=[
            pltpu.VMEM((n_chunks, CHUNK), jnp.int32),
            pltpu.VMEM((b_per_w, D), jnp.float32),
            pltpu.SemaphoreType.DMA,
        ],
        compiler_params=pltpu.CompilerParams(use_tc_tiling_on_sc=False),
    )
    def k(idx_hbm, table_hbm, out_hbm, idx_v, rows_v, sem):
        wid = lax.axis_index("s") * nc + lax.axis_index("c")
        base = wid * b_per_w
        for j in range(n_chunks):
            pltpu.sync_copy(idx_hbm.at[pl.ds(base + j * CHUNK, CHUNK)], idx_v.at[j])
        copies = []
        for j in range(n_chunks):
            copies.append(
                pltpu.async_copy(
                    table_hbm.at[idx_v.at[j]],
                    rows_v.at[pl.ds(j * CHUNK, CHUNK)],
                    sem,
                )
            )
        for c in copies:
            c.wait()
        pltpu.sync_copy(rows_v, out_hbm.at[pl.ds(base, b_per_w)])

    return k


def kernel(indices, table):
    (B,) = indices.shape
    V, D = table.shape
    nc, ns = _sc_geometry()
    return _build(B, V, D, nc, ns)(indices.astype(jnp.int32), table)


# speedup vs baseline: 3.0545x; 3.0545x over previous
import functools

import jax
import jax.numpy as jnp
from jax import lax
from jax.experimental import pallas as pl
from jax.experimental.pallas import tpu as pltpu
from jax.experimental.pallas import tpu_sc as plsc

RING = 8


def _sc_geometry():
    try:
        info = plsc.get_sparse_core_info()
        return info.num_cores, info.num_subcores
    except Exception:
        return 2, 16


@functools.lru_cache(maxsize=None)
def _build(B, V, D, nc, ns):
    nw = nc * ns
    b_per_w = B // nw
    n_groups = b_per_w // RING
    mesh = plsc.VectorSubcoreMesh(core_axis_name="c", subcore_axis_name="s")

    @functools.partial(
        pl.kernel,
        mesh=mesh,
        out_type=jax.ShapeDtypeStruct((D, B), jnp.float32),
        scratch_types=[
            pltpu.VMEM((b_per_w + 16,), jnp.int32),
            pltpu.VMEM((RING, D, 128), jnp.float32),
            pltpu.VMEM((D, b_per_w), jnp.float32),
        ]
        + [pltpu.SemaphoreType.DMA] * RING,
        compiler_params=pltpu.CompilerParams(
            use_tc_tiling_on_sc=True, needs_layout_passes=False
        ),
    )
    def k(idx_hbm, tab_t_hbm, out_t_hbm, idx_vm, ring_v, osg_v, *sems):
        wid = lax.axis_index("s") * nc + lax.axis_index("c")
        base = wid * b_per_w
        pltpu.sync_copy(idx_hbm.at[pl.ds(base, b_per_w)], idx_vm.at[pl.ds(0, b_per_w)])

        rows16 = [lax.iota(jnp.int32, 16) + 16 * t for t in range(D // 16)]

        def fetch(i, r):
            col = pl.multiple_of((i >> 7) << 7, 128)
            pltpu.async_copy(
                tab_t_hbm.at[:, pl.ds(col, 128)], ring_v.at[r], sems[r]
            )

        v0 = idx_vm[pl.ds(0, 16)]
        for r in range(RING):
            fetch(v0[r], r)

        def body(o, _):
            v = idx_vm[pl.ds(o * RING, 16)]
            for r in range(RING):
                pltpu.make_async_copy(
                    tab_t_hbm.at[:, pl.ds(0, 128)], ring_v.at[r], sems[r]
                ).wait()
                lane = jnp.full((16,), v[r] & 127, jnp.int32)
                kk = jnp.full((16,), o * RING + r, jnp.int32)
                for t in range(D // 16):
                    g = plsc.load_gather(ring_v.at[r], [rows16[t], lane])
                    plsc.store_scatter(osg_v, [rows16[t], kk], g)

                @pl.when(o < n_groups - 1)
                def _():
                    fetch(v[RING + r], r)

            return 0

        lax.fori_loop(0, n_groups, body, 0)
        pltpu.sync_copy(osg_v, out_t_hbm.at[:, pl.ds(base, b_per_w)])

    return k


def kernel(indices, table):
    (B,) = indices.shape
    V, D = table.shape
    nc, ns = _sc_geometry()
    out_t = _build(B, V, D, nc, ns)(indices.astype(jnp.int32), table.T)
    return out_t.T


# speedup vs baseline: 3.3910x; 1.1102x over previous
import functools

import jax
import jax.numpy as jnp
from jax import lax
from jax.experimental import pallas as pl
from jax.experimental.pallas import tpu as pltpu
from jax.experimental.pallas import tpu_sc as plsc

CAP = 768
BCOLS = 6
CHUNK = 128


def _sc_geometry():
    try:
        info = plsc.get_sparse_core_info()
        return info.num_cores, info.num_subcores
    except Exception:
        return 2, 16


@functools.lru_cache(maxsize=None)
def _build_k1(B, V, D, nc, ns):
    nw = nc * ns
    ncols = (V + 127) // 128
    base_len = ncols // nw
    rem = ncols - base_len * nw
    n_batches = (base_len + rem + BCOLS - 1) // BCOLS
    max_c0 = ncols - BCOLS
    lanes = BCOLS * 128
    mesh = plsc.VectorSubcoreMesh(core_axis_name="c", subcore_axis_name="s")
    i32 = jnp.int32

    @functools.partial(
        pl.kernel,
        mesh=mesh,
        out_type=(
            jax.ShapeDtypeStruct((nw * CAP, D), jnp.float32),
            jax.ShapeDtypeStruct((nw * CAP,), i32),
        ),
        scratch_types=[
            pltpu.VMEM((B,), i32),
            pltpu.VMEM((CAP + 32,), i32),
            pltpu.VMEM((CAP + 32,), i32),
            pltpu.VMEM((CAP + 32,), i32),
            pltpu.VMEM((CAP + 32,), i32),
            pltpu.VMEM((2, D, lanes), jnp.float32),
            pltpu.VMEM((8, D), jnp.float32),
            pltpu.VMEM((CAP + 16,), i32),
            pltpu.SemaphoreType.DMA,
            pltpu.SemaphoreType.DMA,
        ],
        compiler_params=pltpu.CompilerParams(
            use_tc_tiling_on_sc=True, needs_layout_passes=False
        ),
    )
    def k1(idx_hbm, tab_t_hbm, g_hbm, p_hbm, idx_vm, wl_i, wl_k, mb_i, mb_k,
           ring_v, stag_v, pbuf_v, sem0, sem1):
        sems = (sem0, sem1)
        wid = lax.axis_index("s") * nc + lax.axis_index("c")
        lo = wid * base_len + jnp.minimum(wid, rem)
        hi = lo + base_len + jnp.where(wid < rem, 1, 0)
        gbase = wid * CAP

        iota16 = lax.iota(i32, 16)
        rows16 = [iota16 + 16 * t for t in range(D // 16)]
        lane0 = iota16 == 0

        def c0_of(b):
            return jnp.minimum(lo + b * BCOLS, max_c0)

        def fetch(b, s):
            off = pl.multiple_of(c0_of(b) * 128, 128)
            pltpu.async_copy(
                tab_t_hbm.at[:, pl.ds(off, lanes)], ring_v.at[s], sems[s]
            )

        fetch(0, 0)
        fetch(1, 1)

        pltpu.sync_copy(idx_hbm, idx_vm)

        def scan_body(j, n):
            v = idx_vm[pl.ds(j * 16, 16)]
            c = v >> 7
            m = (c >= lo) & (c < hi)
            plsc.store_compressed(wl_i.at[pl.ds(n, 16)], v, mask=m)
            plsc.store_compressed(wl_k.at[pl.ds(n, 16)], iota16 + j * 16, mask=m)
            return n + plsc.all_reduce_population_count(m)[0]

        m_total = lax.fori_loop(0, B // 16, scan_body, jnp.int32(0))
        n_wl_vecs = (CAP + 32) // 16

        def do_batch(b, s, carry):
            nstage, k_last = carry
            blo = lo + b * BCOLS
            bhi = jnp.minimum(blo + BCOLS, hi)
            c0e = c0_of(b)
            pltpu.make_async_copy(
                tab_t_hbm.at[:, pl.ds(0, lanes)], ring_v.at[s], sems[s]
            ).wait()

            def rescan(j, cnt):
                v = wl_i[pl.ds(j * 16, 16)]
                kv = wl_k[pl.ds(j * 16, 16)]
                pos = iota16 + j * 16
                c = v >> 7
                m = (pos < m_total) & (c >= blo) & (c < bhi)
                plsc.store_compressed(mb_i.at[pl.ds(cnt, 16)], v, mask=m)
                plsc.store_compressed(mb_k.at[pl.ds(cnt, 16)], kv, mask=m)
                return cnt + plsc.all_reduce_population_count(m)[0]

            m_b = lax.fori_loop(0, n_wl_vecs, rescan, jnp.int32(0))

            def extract(e, carry2):
                nst, _kl = carry2
                ev = jnp.full((16,), e, i32)
                iv = plsc.load_gather(mb_i, [ev])
                kv = plsc.load_gather(mb_k, [ev])
                lwb = ((iv >> 7) - c0e) * 128 + (iv & 127)
                r_in = jnp.full((16,), nst & 7, i32)
                for t in range(D // 16):
                    g = plsc.load_gather(ring_v.at[s], [rows16[t], lwb])
                    plsc.store_scatter(stag_v, [r_in, rows16[t]], g)
                plsc.store_scatter(
                    pbuf_v, [jnp.full((16,), nst, i32)], kv, mask=lane0
                )

                @pl.when((nst & 7) == 7)
                def _():
                    goff = pl.multiple_of(gbase + (nst & ~7), 8)
                    pltpu.sync_copy(stag_v, g_hbm.at[pl.ds(goff, 8)])

                return nst + 1, kv[0]

            nstage, k_last = lax.fori_loop(
                0, m_b, extract, (nstage, k_last)
            )

            @pl.when(b + 2 < n_batches)
            def _():
                fetch(b + 2, s)

            return nstage, k_last

        def pair_body(o, carry):
            carry = do_batch(o * 2, 0, carry)
            carry = do_batch(o * 2 + 1, 1, carry)
            return carry

        assert n_batches % 2 == 0
        nstage, k_last = lax.fori_loop(
            0, n_batches // 2, pair_body, (jnp.int32(0), jnp.int32(0))
        )

        r_last = jnp.full((16,), jnp.maximum(nstage - 1, 0) & 7, i32)
        gdup = [plsc.load_gather(stag_v, [r_last, rows16[t]])
                for t in range(D // 16)]
        for rr in range(8):
            @pl.when(rr >= (nstage & 7))
            def _(rr=rr):
                for t in range(D // 16):
                    plsc.store_scatter(
                        stag_v, [jnp.full((16,), rr, i32), rows16[t]], gdup[t]
                    )

        def flush_pad(nb, _):
            goff = pl.multiple_of(gbase + nb * 8, 8)
            pltpu.sync_copy(stag_v, g_hbm.at[pl.ds(goff, 8)])
            return 0

        lax.fori_loop(nstage >> 3, CAP // 8, flush_pad, 0)

        def pad_p(j, _):
            v = pbuf_v[pl.ds(j * 16, 16)]
            pos = iota16 + j * 16
            v2 = jnp.where(pos < nstage, v, jnp.full((16,), k_last, i32))
            pbuf_v[pl.ds(j * 16, 16)] = v2
            return 0

        lax.fori_loop(0, CAP // 16, pad_p, 0)
        pltpu.sync_copy(
            pbuf_v.at[pl.ds(0, CAP)], p_hbm.at[pl.ds(gbase, CAP)]
        )

    return k1


@functools.lru_cache(maxsize=None)
def _build_k2(B, D, nc, ns):
    nw = nc * ns
    n_chunks = CAP // CHUNK
    mesh = plsc.VectorSubcoreMesh(core_axis_name="c", subcore_axis_name="s")

    @functools.partial(
        pl.kernel,
        mesh=mesh,
        out_type=jax.ShapeDtypeStruct((B, D), jnp.float32),
        scratch_types=[
            pltpu.VMEM((n_chunks, CHUNK), jnp.int32),
            pltpu.VMEM((CAP, D), jnp.float32),
            pltpu.SemaphoreType.DMA,
        ],
        compiler_params=pltpu.CompilerParams(use_tc_tiling_on_sc=False),
    )
    def k2(g_hbm, p_hbm, out_hbm, p_v, rows_v, sem):
        wid = lax.axis_index("s") * nc + lax.axis_index("c")
        gbase = wid * CAP
        for j in range(n_chunks):
            pltpu.sync_copy(
                p_hbm.at[pl.ds(gbase + j * CHUNK, CHUNK)], p_v.at[j]
            )
        pltpu.sync_copy(g_hbm.at[pl.ds(gbase, CAP)], rows_v)
        copies = []
        for j in range(n_chunks):
            copies.append(
                pltpu.async_copy(
                    rows_v.at[pl.ds(j * CHUNK, CHUNK)],
                    out_hbm.at[p_v.at[j]],
                    sem,
                )
            )
        for c in copies:
            c.wait()

    return k2


def kernel(indices, table):
    (B,) = indices.shape
    V, D = table.shape
    nc, ns = _sc_geometry()
    g, p = _build_k1(B, V, D, nc, ns)(indices.astype(jnp.int32), table.T)
    return _build_k2(B, D, nc, ns)(g, p)


# speedup vs baseline: 3.7045x; 1.0924x over previous
import functools

import jax
import jax.numpy as jnp
from jax import lax
from jax.experimental import pallas as pl
from jax.experimental.pallas import tpu as pltpu
from jax.experimental.pallas import tpu_sc as plsc

CAP = 768
BCOLS = 4
CHUNK = 128


def _sc_geometry():
    try:
        info = plsc.get_sparse_core_info()
        return info.num_cores, info.num_subcores
    except Exception:
        return 2, 16


@functools.lru_cache(maxsize=None)
def _build_k1(B, V, D, nc, ns):
    nw = nc * ns
    ncols = (V + 127) // 128
    base_len = ncols // nw
    rem = ncols - base_len * nw
    n_batches = (base_len + rem + BCOLS - 1) // BCOLS
    max_c0 = ncols - BCOLS
    lanes = BCOLS * 128
    mesh = plsc.VectorSubcoreMesh(core_axis_name="c", subcore_axis_name="s")
    i32 = jnp.int32

    @functools.partial(
        pl.kernel,
        mesh=mesh,
        out_type=(
            jax.ShapeDtypeStruct((nw * CAP, D), jnp.float32),
            jax.ShapeDtypeStruct((nw * CAP,), i32),
        ),
        scratch_types=[
            pltpu.VMEM((B,), i32),
            pltpu.VMEM((CAP + 32,), i32),
            pltpu.VMEM((CAP + 32,), i32),
            pltpu.VMEM((CAP + 32,), i32),
            pltpu.VMEM((CAP + 32,), i32),
            pltpu.VMEM((3, D, lanes), jnp.float32),
            pltpu.VMEM((8, D), jnp.float32),
            pltpu.VMEM((CAP + 16,), i32),
            pltpu.SemaphoreType.DMA,
            pltpu.SemaphoreType.DMA,
            pltpu.SemaphoreType.DMA,
        ],
        compiler_params=pltpu.CompilerParams(
            use_tc_tiling_on_sc=True, needs_layout_passes=False
        ),
    )
    def k1(idx_hbm, tab_t_hbm, g_hbm, p_hbm, idx_vm, wl_i, wl_k, mb_i, mb_k,
           ring_v, stag_v, pbuf_v, sem0, sem1, sem2):
        sems = (sem0, sem1, sem2)
        wid = lax.axis_index("s") * nc + lax.axis_index("c")
        lo = wid * base_len + jnp.minimum(wid, rem)
        hi = lo + base_len + jnp.where(wid < rem, 1, 0)
        gbase = wid * CAP

        iota16 = lax.iota(i32, 16)
        rows16 = [iota16 + 16 * t for t in range(D // 16)]
        lane0 = iota16 == 0

        def c0_of(b):
            return jnp.minimum(lo + b * BCOLS, max_c0)

        def fetch(b, s):
            off = pl.multiple_of(c0_of(b) * 128, 128)
            pltpu.async_copy(
                tab_t_hbm.at[:, pl.ds(off, lanes)], ring_v.at[s], sems[s]
            )

        fetch(0, 0)
        fetch(1, 1)
        fetch(2, 2)

        pltpu.sync_copy(idx_hbm, idx_vm)

        def scan_body(j, n):
            v = idx_vm[pl.ds(j * 16, 16)]
            c = v >> 7
            m = (c >= lo) & (c < hi)
            plsc.store_compressed(wl_i.at[pl.ds(n, 16)], v, mask=m)
            plsc.store_compressed(wl_k.at[pl.ds(n, 16)], iota16 + j * 16, mask=m)
            return n + plsc.all_reduce_population_count(m)[0]

        m_total = lax.fori_loop(0, B // 16, scan_body, jnp.int32(0))
        n_wl_vecs = (CAP + 32) // 16

        def do_batch(b, s, carry):
            nstage, k_last = carry
            blo = lo + b * BCOLS
            bhi = jnp.minimum(blo + BCOLS, hi)
            c0e = c0_of(b)
            pltpu.make_async_copy(
                tab_t_hbm.at[:, pl.ds(0, lanes)], ring_v.at[s], sems[s]
            ).wait()

            def rescan(j, cnt):
                v = wl_i[pl.ds(j * 16, 16)]
                kv = wl_k[pl.ds(j * 16, 16)]
                pos = iota16 + j * 16
                c = v >> 7
                m = (pos < m_total) & (c >= blo) & (c < bhi)
                plsc.store_compressed(mb_i.at[pl.ds(cnt, 16)], v, mask=m)
                plsc.store_compressed(mb_k.at[pl.ds(cnt, 16)], kv, mask=m)
                return cnt + plsc.all_reduce_population_count(m)[0]

            m_b = lax.fori_loop(0, n_wl_vecs, rescan, jnp.int32(0))

            def extract(e, carry2):
                nst, _kl = carry2
                ev = jnp.full((16,), e, i32)
                iv = plsc.load_gather(mb_i, [ev])
                kv = plsc.load_gather(mb_k, [ev])
                lwb = ((iv >> 7) - c0e) * 128 + (iv & 127)
                r_in = jnp.full((16,), nst & 7, i32)
                for t in range(D // 16):
                    g = plsc.load_gather(ring_v.at[s], [rows16[t], lwb])
                    plsc.store_scatter(stag_v, [r_in, rows16[t]], g)
                plsc.store_scatter(
                    pbuf_v, [jnp.full((16,), nst, i32)], kv, mask=lane0
                )

                @pl.when((nst & 7) == 7)
                def _():
                    goff = pl.multiple_of(gbase + (nst & ~7), 8)
                    pltpu.sync_copy(stag_v, g_hbm.at[pl.ds(goff, 8)])

                return nst + 1, kv[0]

            nstage, k_last = lax.fori_loop(
                0, m_b, extract, (nstage, k_last)
            )

            @pl.when(b + 3 < n_batches)
            def _():
                fetch(b + 3, s)

            return nstage, k_last

        def trip_body(o, carry):
            carry = do_batch(o * 3, 0, carry)
            carry = do_batch(o * 3 + 1, 1, carry)
            carry = do_batch(o * 3 + 2, 2, carry)
            return carry

        assert n_batches % 3 == 0
        nstage, k_last = lax.fori_loop(
            0, n_batches // 3, trip_body, (jnp.int32(0), jnp.int32(0))
        )

        r_last = jnp.full((16,), jnp.maximum(nstage - 1, 0) & 7, i32)
        gdup = [plsc.load_gather(stag_v, [r_last, rows16[t]])
                for t in range(D // 16)]
        for rr in range(8):
            @pl.when(rr >= (nstage & 7))
            def _(rr=rr):
                for t in range(D // 16):
                    plsc.store_scatter(
                        stag_v, [jnp.full((16,), rr, i32), rows16[t]], gdup[t]
                    )

        def flush_pad(nb, _):
            goff = pl.multiple_of(gbase + nb * 8, 8)
            pltpu.sync_copy(stag_v, g_hbm.at[pl.ds(goff, 8)])
            return 0

        lax.fori_loop(nstage >> 3, CAP // 8, flush_pad, 0)

        def pad_p(j, _):
            v = pbuf_v[pl.ds(j * 16, 16)]
            pos = iota16 + j * 16
            v2 = jnp.where(pos < nstage, v, jnp.full((16,), k_last, i32))
            pbuf_v[pl.ds(j * 16, 16)] = v2
            return 0

        lax.fori_loop(0, CAP // 16, pad_p, 0)
        pltpu.sync_copy(
            pbuf_v.at[pl.ds(0, CAP)], p_hbm.at[pl.ds(gbase, CAP)]
        )

    return k1


@functools.lru_cache(maxsize=None)
def _build_k2(B, D, nc, ns):
    nw = nc * ns
    n_chunks = CAP // CHUNK
    mesh = plsc.VectorSubcoreMesh(core_axis_name="c", subcore_axis_name="s")

    @functools.partial(
        pl.kernel,
        mesh=mesh,
        out_type=jax.ShapeDtypeStruct((B, D), jnp.float32),
        scratch_types=[
            pltpu.VMEM((n_chunks, CHUNK), jnp.int32),
            pltpu.VMEM((CAP, D), jnp.float32),
            pltpu.SemaphoreType.DMA,
        ],
        compiler_params=pltpu.CompilerParams(use_tc_tiling_on_sc=False),
    )
    def k2(g_hbm, p_hbm, out_hbm, p_v, rows_v, sem):
        wid = lax.axis_index("s") * nc + lax.axis_index("c")
        gbase = wid * CAP
        for j in range(n_chunks):
            pltpu.sync_copy(
                p_hbm.at[pl.ds(gbase + j * CHUNK, CHUNK)], p_v.at[j]
            )
        pltpu.sync_copy(g_hbm.at[pl.ds(gbase, CAP)], rows_v)
        copies = []
        for j in range(n_chunks):
            copies.append(
                pltpu.async_copy(
                    rows_v.at[pl.ds(j * CHUNK, CHUNK)],
                    out_hbm.at[p_v.at[j]],
                    sem,
                )
            )
        for c in copies:
            c.wait()

    return k2


def kernel(indices, table):
    (B,) = indices.shape
    V, D = table.shape
    nc, ns = _sc_geometry()
    g, p = _build_k1(B, V, D, nc, ns)(indices.astype(jnp.int32), table.T)
    return _build_k2(B, D, nc, ns)(g, p)
